# Initial kernel scaffold; baseline (speedup 1.0000x reference)
import functools

import jax
import jax.numpy as jnp
from jax import lax
from jax.experimental import pallas as pl
from jax.experimental.pallas import tpu as pltpu
from jax.experimental.pallas import tpu_sc as plsc

N_NODES = 10000
N_EDGES = 320000
DIM = 128
RW = 144

NUM_CORES = 2
NUM_SUBCORES = 16
NUM_TILES = NUM_CORES * NUM_SUBCORES
EDGES_PER_TILE = N_EDGES // NUM_TILES
CHUNK = 128
NFULL = EDGES_PER_TILE // CHUNK
TAIL = EDGES_PER_TILE - NFULL * CHUNK
ROWS_PER_SUB = N_NODES // NUM_SUBCORES


def _xform_body(x_ref, w_ref, as_ref, ad_ref, h_ref, asrc_ref, adst_ref):
    h = jnp.dot(x_ref[...], w_ref[...], preferred_element_type=jnp.float32)
    h_ref[...] = h
    asrc_ref[...] = jnp.sum(h * as_ref[...][None, :], axis=1)
    adst_ref[...] = jnp.sum(h * ad_ref[...][None, :], axis=1)


def _combine_body(p_ref, b_ref, pw_ref, o_ref):
    p = p_ref[0] + p_ref[1]
    msg = p[:, :DIM]
    den = p[:, DIM:DIM + 1]
    o = msg / (den + 1e-16) + b_ref[...][None, :]
    o_ref[...] = jnp.maximum(o, 0.0) + pw_ref[...][None, :] * jnp.minimum(o, 0.0)


def _sc_edge_kernel(src, dst, h, asrc, adst):
    mesh = plsc.VectorSubcoreMesh(core_axis_name="c", subcore_axis_name="s")

    @functools.partial(
        pl.kernel,
        out_type=jax.ShapeDtypeStruct((NUM_CORES, N_NODES, RW), jnp.float32),
        mesh=mesh,
        scratch_types=[
            pltpu.VMEM((N_NODES,), jnp.float32),
            pltpu.VMEM((N_NODES,), jnp.float32),
            pltpu.VMEM((CHUNK,), jnp.int32),
            pltpu.VMEM((CHUNK,), jnp.int32),
            pltpu.VMEM((TAIL,), jnp.int32),
            pltpu.VMEM((TAIL,), jnp.int32),
            pltpu.VMEM((CHUNK,), jnp.float32),
            pltpu.VMEM((CHUNK, DIM), jnp.float32),
            pltpu.VMEM((CHUNK, RW), jnp.float32),
            pltpu.VMEM_SHARED((N_NODES, RW), jnp.float32),
            pltpu.SemaphoreType.DMA,
        ],
    )
    def body(src_hbm, dst_hbm, h_hbm, asrc_hbm, adst_hbm, out_hbm,
             asrc_v, adst_v, src_v, dst_v, src_t, dst_t, w_v, rows_v,
             stage_v, acc, gsem):
        c = lax.axis_index("c")
        s = lax.axis_index("s")
        wid = s * NUM_CORES + c
        ebase = wid * EDGES_PER_TILE

        pltpu.sync_copy(asrc_hbm, asrc_v)
        pltpu.sync_copy(adst_hbm, adst_v)

        zero16 = jnp.zeros((16,), jnp.float32)

        @pl.loop(0, CHUNK)
        def _(r):
            for t in range(RW // 16):
                stage_v[r, pl.ds(t * 16, 16)] = zero16

        @pl.loop(0, 5)
        def _(i):
            pltpu.sync_copy(stage_v.at[pl.ds(0, 125)],
                            acc.at[pl.ds(s * ROWS_PER_SUB + i * 125, 125)])

        plsc.subcore_barrier()

        iot = lax.iota(jnp.int32, 16)

        def process(base, k, sv, dv):
            pltpu.sync_copy(src_hbm.at[pl.ds(base, k)], sv)
            pltpu.sync_copy(dst_hbm.at[pl.ds(base, k)], dv)
            gcopy = pltpu.async_copy(h_hbm.at[sv], rows_v.at[pl.ds(0, k)], gsem)

            @pl.loop(0, k // 16)
            def _(j):
                si = sv[pl.ds(j * 16, 16)]
                di = dv[pl.ds(j * 16, 16)]
                e = (plsc.load_gather(asrc_v, [si])
                     + plsc.load_gather(adst_v, [di]))
                e = jnp.where(e > 0, e, e * jnp.float32(0.2))
                w_v[pl.ds(j * 16, 16)] = jnp.exp(e)

            gcopy.wait()

            @pl.loop(0, k)
            def _(r):
                wv = plsc.load_gather(w_v, [jnp.full((16,), r, jnp.int32)])
                for t in range(DIM // 16):
                    stage_v[r, pl.ds(t * 16, 16)] = (
                        rows_v[r, pl.ds(t * 16, 16)] * wv)
                stage_v[r, pl.ds(DIM, 16)] = jnp.where(
                    iot == 0, wv, jnp.float32(0.0))

            pltpu.sync_copy(stage_v.at[pl.ds(0, k)], acc.at[dv], add=True)

        @pl.loop(0, NFULL)
        def _(i):
            process(ebase + i * CHUNK, CHUNK, src_v, dst_v)

        process(ebase + NFULL * CHUNK, TAIL, src_t, dst_t)

        plsc.subcore_barrier()
        pltpu.sync_copy(acc.at[pl.ds(s * ROWS_PER_SUB, ROWS_PER_SUB)],
                        out_hbm.at[c, pl.ds(s * ROWS_PER_SUB, ROWS_PER_SUB)])

    return body(src, dst, h, asrc, adst)


def kernel(x, edge_index, W, att_src, att_dst, bias, prelu_w):
    src = edge_index[0].astype(jnp.int32)
    dst = edge_index[1].astype(jnp.int32)
    as_vec = att_src.reshape(DIM)
    ad_vec = att_dst.reshape(DIM)

    h, asrc, adst = pl.pallas_call(
        _xform_body,
        out_shape=(
            jax.ShapeDtypeStruct((N_NODES, DIM), jnp.float32),
            jax.ShapeDtypeStruct((N_NODES,), jnp.float32),
            jax.ShapeDtypeStruct((N_NODES,), jnp.float32),
        ),
    )(x, W, as_vec, ad_vec)

    parts = _sc_edge_kernel(src, dst, h, asrc, adst)

    out = pl.pallas_call(
        _combine_body,
        out_shape=jax.ShapeDtypeStruct((N_NODES, DIM), jnp.float32),
    )(parts, bias, prelu_w)
    return out


# speedup vs baseline: 13.8067x; 13.8067x over previous
import functools

import jax
import jax.numpy as jnp
from jax import lax
from jax.experimental import pallas as pl
from jax.experimental.pallas import tpu as pltpu
from jax.experimental.pallas import tpu_sc as plsc

N_NODES = 10000
N_EDGES = 320000
DIM = 128
RW = 144

NUM_CORES = 2
NUM_SUBCORES = 16
NUM_TILES = NUM_CORES * NUM_SUBCORES
EDGES_PER_TILE = N_EDGES // NUM_TILES
CHUNK = 64
NFULL = EDGES_PER_TILE // CHUNK
TAIL = EDGES_PER_TILE - NFULL * CHUNK
ROWS_PER_SUB = N_NODES // NUM_SUBCORES


def _xform_body(x_ref, w_ref, as_ref, ad_ref, h_ref, asrc_ref, adst_ref):
    h = jnp.dot(x_ref[...], w_ref[...], preferred_element_type=jnp.float32)
    h_ref[...] = h
    asrc_ref[...] = jnp.sum(h * as_ref[...][None, :], axis=1)
    adst_ref[...] = jnp.sum(h * ad_ref[...][None, :], axis=1)


def _combine_body(p_ref, b_ref, pw_ref, o_ref):
    p = p_ref[0] + p_ref[1]
    msg = p[:, :DIM]
    den = p[:, DIM:DIM + 1]
    o = msg / (den + 1e-16) + b_ref[...][None, :]
    o_ref[...] = jnp.maximum(o, 0.0) + pw_ref[...][None, :] * jnp.minimum(o, 0.0)


def _sc_edge_kernel(src, dst, h, asrc, adst):
    mesh = plsc.VectorSubcoreMesh(core_axis_name="c", subcore_axis_name="s")

    @functools.partial(
        pl.kernel,
        out_type=jax.ShapeDtypeStruct((NUM_CORES, N_NODES, RW), jnp.float32),
        mesh=mesh,
        compiler_params=pltpu.CompilerParams(
            use_tc_tiling_on_sc=False, needs_layout_passes=False),
        scratch_types=[
            pltpu.VMEM((N_NODES,), jnp.float32),
            pltpu.VMEM((N_NODES,), jnp.float32),
            pltpu.VMEM((CHUNK,), jnp.int32),
            pltpu.VMEM((CHUNK,), jnp.int32),
            pltpu.VMEM((TAIL,), jnp.int32),
            pltpu.VMEM((TAIL,), jnp.int32),
            pltpu.VMEM((CHUNK,), jnp.float32),
            pltpu.VMEM((CHUNK, DIM), jnp.float32),
            pltpu.VMEM((CHUNK, RW), jnp.float32),
            pltpu.VMEM_SHARED((N_NODES, RW), jnp.float32),
            pltpu.SemaphoreType.DMA,
        ],
    )
    def body(src_hbm, dst_hbm, h_hbm, asrc_hbm, adst_hbm, out_hbm,
             asrc_v, adst_v, src_v, dst_v, src_t, dst_t, w_v, rows_v,
             stage_v, acc, gsem):
        c = lax.axis_index("c")
        s = lax.axis_index("s")
        wid = s * NUM_CORES + c
        ebase = wid * EDGES_PER_TILE

        pltpu.sync_copy(asrc_hbm, asrc_v)
        pltpu.sync_copy(adst_hbm, adst_v)

        zero16 = jnp.zeros((16,), jnp.float32)

        @pl.loop(0, CHUNK)
        def _(r):
            for t in range(RW // 16):
                stage_v[r, pl.ds(t * 16, 16)] = zero16

        @pl.loop(0, 9)
        def _(i):
            pltpu.sync_copy(stage_v.at[pl.ds(0, CHUNK)],
                            acc.at[pl.ds(s * ROWS_PER_SUB + i * CHUNK, CHUNK)])
        pltpu.sync_copy(stage_v.at[pl.ds(0, ROWS_PER_SUB - 9 * CHUNK)],
                        acc.at[pl.ds(s * ROWS_PER_SUB + 9 * CHUNK,
                                     ROWS_PER_SUB - 9 * CHUNK)])

        plsc.subcore_barrier()

        iot = lax.iota(jnp.int32, 16)

        def process(base, k, sv, dv):
            pltpu.sync_copy(src_hbm.at[pl.ds(base, k)], sv)
            pltpu.sync_copy(dst_hbm.at[pl.ds(base, k)], dv)
            gcopy = pltpu.async_copy(h_hbm.at[sv], rows_v.at[pl.ds(0, k)], gsem)

            @pl.loop(0, k // 16)
            def _(j):
                si = sv[pl.ds(j * 16, 16)]
                di = dv[pl.ds(j * 16, 16)]
                e = (plsc.load_gather(asrc_v, [si])
                     + plsc.load_gather(adst_v, [di]))
                e = jnp.where(e > 0, e, e * jnp.float32(0.2))
                w_v[pl.ds(j * 16, 16)] = jnp.exp(e)

            gcopy.wait()

            @pl.loop(0, k)
            def _(r):
                wv = plsc.load_gather(w_v, [jnp.full((16,), r, jnp.int32)])
                for t in range(DIM // 16):
                    stage_v[r, pl.ds(t * 16, 16)] = (
                        rows_v[r, pl.ds(t * 16, 16)] * wv)
                stage_v[r, pl.ds(DIM, 16)] = jnp.where(
                    iot == 0, wv, jnp.float32(0.0))

            pltpu.sync_copy(stage_v.at[pl.ds(0, k)], acc.at[dv], add=True)

        @pl.loop(0, NFULL)
        def _(i):
            process(ebase + i * CHUNK, CHUNK, src_v, dst_v)

        process(ebase + NFULL * CHUNK, TAIL, src_t, dst_t)

        plsc.subcore_barrier()
        pltpu.sync_copy(acc.at[pl.ds(s * ROWS_PER_SUB, ROWS_PER_SUB)],
                        out_hbm.at[c, pl.ds(s * ROWS_PER_SUB, ROWS_PER_SUB)])

    return body(src, dst, h, asrc, adst)


def kernel(x, edge_index, W, att_src, att_dst, bias, prelu_w):
    src = edge_index[0].astype(jnp.int32)
    dst = edge_index[1].astype(jnp.int32)
    as_vec = att_src.reshape(DIM)
    ad_vec = att_dst.reshape(DIM)

    h, asrc, adst = pl.pallas_call(
        _xform_body,
        out_shape=(
            jax.ShapeDtypeStruct((N_NODES, DIM), jnp.float32),
            jax.ShapeDtypeStruct((N_NODES,), jnp.float32),
            jax.ShapeDtypeStruct((N_NODES,), jnp.float32),
        ),
    )(x, W, as_vec, ad_vec)

    parts = _sc_edge_kernel(src, dst, h, asrc, adst)

    out = pl.pallas_call(
        _combine_body,
        out_shape=jax.ShapeDtypeStruct((N_NODES, DIM), jnp.float32),
    )(parts, bias, prelu_w)
    return out
